# Initial kernel scaffold; baseline (speedup 1.0000x reference)
import jax
import jax.numpy as jnp
from jax.experimental import pallas as pl


def kernel(x, W):
    raise NotImplementedError("write your pallas kernel here")


# speedup vs baseline: 1.1922x; 1.1922x over previous
import functools

import jax
import jax.numpy as jnp
from jax import lax
from jax.experimental import pallas as pl
from jax.experimental.pallas import tpu as pltpu
from jax.experimental.pallas import tpu_sc as plsc

_K = 8192
_D = 64
_TM = 256
_LOSS_COEFF = 1.25


def _dist_argmin_body(z_ref, wt_ref, idx_ref, loss_ref):
    t = pl.program_id(0)
    nt = pl.num_programs(0)
    z = z_ref[...]
    wt = wt_ref[...]
    mm = lax.dot_general(z, wt, (((1,), (0,)), ((), ())),
                         preferred_element_type=jnp.float32)
    z2 = jnp.sum(z * z, axis=1, keepdims=True)
    w2 = jnp.sum(wt * wt, axis=0, keepdims=True)
    dist = (z2 + w2) - 2.0 * mm
    h = _K // 2
    d0, d1 = dist[:, :h], dist[:, h:]
    cols = lax.broadcasted_iota(jnp.int32, d0.shape, 1)
    m0 = jnp.min(d0, axis=1, keepdims=True)
    i0 = jnp.min(jnp.where(d0 == m0, cols, jnp.int32(_K)), axis=1)
    m1 = jnp.min(d1, axis=1, keepdims=True)
    i1 = jnp.min(jnp.where(d1 == m1, cols, jnp.int32(_K)), axis=1) + h
    q0 = m0.astype(jnp.bfloat16).astype(jnp.float32)
    take = (m1 < q0)[:, 0]
    idx = jnp.where(take, i1, i0)
    vals = jnp.where(take, m1[:, 0], m0[:, 0])
    idx_ref[0, 0, :] = idx
    prev = loss_ref[...]
    acc = jnp.where(t == 0, 0.0, prev[0, 0]) + jnp.sum(vals)
    n_elems = nt * _TM * _D
    out = jnp.where(t == nt - 1, acc * (_LOSS_COEFF / n_elems), acc)
    loss_ref[...] = out.reshape(1, 1)


def _dist_argmin(z, wt):
    n = z.shape[0]
    nt = n // _TM
    return pl.pallas_call(
        _dist_argmin_body,
        grid=(nt,),
        in_specs=[
            pl.BlockSpec((_TM, _D), lambda t: (t, 0)),
            pl.BlockSpec((_D, _K), lambda t: (0, 0)),
        ],
        out_specs=[
            pl.BlockSpec((1, 1, _TM), lambda t: (t, 0, 0)),
            pl.BlockSpec((1, 1), lambda t: (0, 0)),
        ],
        out_shape=[
            jax.ShapeDtypeStruct((nt, 1, _TM), jnp.int32),
            jax.ShapeDtypeStruct((1, 1), jnp.float32),
        ],
    )(z, wt)


@functools.lru_cache(maxsize=None)
def _make_sc_gather(b, d):
    info = plsc.get_sparse_core_info()
    nw = info.num_cores * info.num_subcores
    b_per_w = b // nw
    mesh = plsc.VectorSubcoreMesh(core_axis_name="c", subcore_axis_name="s")

    @functools.partial(
        pl.kernel, mesh=mesh,
        out_type=jax.ShapeDtypeStruct((b, d), jnp.float32),
        compiler_params=pltpu.CompilerParams(use_tc_tiling_on_sc=False),
        scratch_types=[
            pltpu.VMEM((b_per_w,), jnp.int32),
            pltpu.VMEM((b_per_w, d), jnp.float32),
            pltpu.SemaphoreType.DMA,
        ],
    )
    def gather_k(table_hbm, idx_hbm, out_hbm, idx_v, rows_v, sem):
        wid = lax.axis_index("s") * info.num_cores + lax.axis_index("c")
        base = wid * b_per_w
        pltpu.sync_copy(idx_hbm.at[pl.ds(base, b_per_w)], idx_v)
        pltpu.async_copy(table_hbm.at[idx_v], rows_v, sem).wait()
        pltpu.sync_copy(rows_v, out_hbm.at[pl.ds(base, b_per_w)])

    return gather_k


def kernel(x, W):
    b, c, h, w = x.shape
    xp = jnp.transpose(x, (0, 2, 3, 1))
    z = xp.reshape(-1, c)
    idx3, loss11 = _dist_argmin(z, W.T)
    idx = idx3.reshape(-1)
    embed = _make_sc_gather(z.shape[0], c)(W, idx)
    embed_out = jnp.transpose(embed.reshape(b, h, w, c), (0, 3, 1, 2))
    return embed_out, loss11[0, 0], idx


# speedup vs baseline: 1.2011x; 1.0075x over previous
import functools

import jax
import jax.numpy as jnp
from jax import lax
from jax.experimental import pallas as pl
from jax.experimental.pallas import tpu as pltpu
from jax.experimental.pallas import tpu_sc as plsc

_K = 8192
_D = 64
_TM = 256
_LOSS_COEFF = 1.25


def _w2_body(w_ref, w2_ref):
    w = w_ref[...]
    ones = jnp.ones((8, _D), jnp.float32)
    ww = lax.dot_general(ones, w * w, (((1,), (1,)), ((), ())),
                         preferred_element_type=jnp.float32)
    w2_ref[...] = ww[:1]


def _w2_row(w):
    return pl.pallas_call(
        _w2_body,
        out_shape=jax.ShapeDtypeStruct((1, _K), jnp.float32),
    )(w)


def _dist_argmin_body(z_ref, w_ref, cols_ref, w2_ref, idx_ref, loss_ref):
    t = pl.program_id(0)
    nt = pl.num_programs(0)
    z = z_ref[...]
    w = w_ref[...]

    mm2 = lax.dot_general(z + z, w, (((1,), (1,)), ((), ())),
                          preferred_element_type=jnp.float32)
    z2 = jnp.sum(z * z, axis=1, keepdims=True)
    w2 = w2_ref[...]
    dist = (z2 + w2) - mm2
    h = _K // 2
    d0, d1 = dist[:, :h], dist[:, h:]
    cols = cols_ref[...]
    big = jnp.float32(_K)
    m0 = jnp.min(d0, axis=1, keepdims=True)
    i0 = jnp.min(jnp.where(d0 == m0, cols, big), axis=1)
    m1 = jnp.min(d1, axis=1, keepdims=True)
    i1 = jnp.min(jnp.where(d1 == m1, cols, big), axis=1) + h
    q0 = m0.astype(jnp.bfloat16).astype(jnp.float32)
    take = (m1 < q0)[:, 0]
    idx = jnp.where(take, i1, i0).astype(jnp.int32)
    vals = jnp.where(take, m1[:, 0], m0[:, 0])
    idx_ref[0, 0, :] = idx
    prev = loss_ref[...]
    acc = jnp.where(t == 0, 0.0, prev[0, 0]) + jnp.sum(vals)
    n_elems = nt * _TM * _D
    out = jnp.where(t == nt - 1, acc * (_LOSS_COEFF / n_elems), acc)
    loss_ref[...] = out.reshape(1, 1)


def _dist_argmin(z, w):
    n = z.shape[0]
    nt = n // _TM
    return pl.pallas_call(
        _dist_argmin_body,
        grid=(nt,),
        in_specs=[
            pl.BlockSpec((_TM, _D), lambda t: (t, 0)),
            pl.BlockSpec((_K, _D), lambda t: (0, 0)),
            pl.BlockSpec((1, _K // 2), lambda t: (0, 0)),
            pl.BlockSpec((1, _K), lambda t: (0, 0)),
        ],
        out_specs=[
            pl.BlockSpec((1, 1, _TM), lambda t: (t, 0, 0)),
            pl.BlockSpec((1, 1), lambda t: (0, 0)),
        ],
        out_shape=[
            jax.ShapeDtypeStruct((nt, 1, _TM), jnp.int32),
            jax.ShapeDtypeStruct((1, 1), jnp.float32),
        ],
    )(z, w, jnp.arange(_K // 2, dtype=jnp.float32).reshape(1, -1), _w2_row(w))


@functools.lru_cache(maxsize=None)
def _make_sc_gather(b, d):
    info = plsc.get_sparse_core_info()
    nw = info.num_cores * info.num_subcores
    b_per_w = b // nw
    mesh = plsc.VectorSubcoreMesh(core_axis_name="c", subcore_axis_name="s")

    @functools.partial(
        pl.kernel, mesh=mesh,
        out_type=jax.ShapeDtypeStruct((b, d), jnp.float32),
        compiler_params=pltpu.CompilerParams(use_tc_tiling_on_sc=False),
        scratch_types=[
            pltpu.VMEM((b_per_w,), jnp.int32),
            pltpu.VMEM((b_per_w, d), jnp.float32),
            pltpu.SemaphoreType.DMA,
        ],
    )
    def gather_k(table_hbm, idx_hbm, out_hbm, idx_v, rows_v, sem):
        wid = lax.axis_index("s") * info.num_cores + lax.axis_index("c")
        base = wid * b_per_w
        pltpu.sync_copy(idx_hbm.at[pl.ds(base, b_per_w)], idx_v)
        pltpu.async_copy(table_hbm.at[idx_v], rows_v, sem).wait()
        pltpu.sync_copy(rows_v, out_hbm.at[pl.ds(base, b_per_w)])

    return gather_k


def kernel(x, W):
    b, c, h, w = x.shape
    xp = jnp.transpose(x, (0, 2, 3, 1))
    z = xp.reshape(-1, c)
    idx3, loss11 = _dist_argmin(z, W)
    idx = idx3.reshape(-1)
    embed = _make_sc_gather(z.shape[0], c)(W, idx)
    embed_out = jnp.transpose(embed.reshape(b, h, w, c), (0, 3, 1, 2))
    return embed_out, loss11[0, 0], idx


# speedup vs baseline: 1.2503x; 1.0410x over previous
import functools

import jax
import jax.numpy as jnp
from jax import lax
from jax.experimental import pallas as pl
from jax.experimental.pallas import tpu as pltpu
from jax.experimental.pallas import tpu_sc as plsc

_K = 8192
_D = 64
_TM = 1024
_LOSS_COEFF = 1.25


def _w2_body(w_ref, w2_ref):
    w = w_ref[...]
    ones = jnp.ones((8, _D), jnp.float32)
    ww = lax.dot_general(ones, w * w, (((1,), (1,)), ((), ())),
                         preferred_element_type=jnp.float32)
    w2_ref[...] = ww[:1]


def _w2_row(w):
    return pl.pallas_call(
        _w2_body,
        out_shape=jax.ShapeDtypeStruct((1, _K), jnp.float32),
    )(w)


def _dist_argmin_body(z_ref, w_ref, cols_ref, w2_ref, idx_ref, loss_ref):
    t = pl.program_id(0)
    nt = pl.num_programs(0)
    z = z_ref[...]
    w = w_ref[...]

    mm2 = lax.dot_general(z + z, w, (((1,), (1,)), ((), ())),
                          preferred_element_type=jnp.float32)
    z2 = jnp.sum(z * z, axis=1, keepdims=True)
    w2 = w2_ref[...]
    dist = (z2 + w2) - mm2
    h = _K // 2
    d0, d1 = dist[:, :h], dist[:, h:]
    cols = cols_ref[...]
    big = jnp.float32(_K)
    m0 = jnp.min(d0, axis=1, keepdims=True)
    i0 = jnp.min(jnp.where(d0 == m0, cols, big), axis=1)
    m1 = jnp.min(d1, axis=1, keepdims=True)
    i1 = jnp.min(jnp.where(d1 == m1, cols, big), axis=1) + h
    q0 = m0.astype(jnp.bfloat16).astype(jnp.float32)
    take = (m1 < q0)[:, 0]
    idx = jnp.where(take, i1, i0).astype(jnp.int32)
    vals = jnp.where(take, m1[:, 0], m0[:, 0])
    idx_ref[0, 0, :] = idx
    prev = loss_ref[...]
    acc = jnp.where(t == 0, 0.0, prev[0, 0]) + jnp.sum(vals)
    n_elems = nt * _TM * _D
    out = jnp.where(t == nt - 1, acc * (_LOSS_COEFF / n_elems), acc)
    loss_ref[...] = out.reshape(1, 1)


def _dist_argmin(z, w):
    n = z.shape[0]
    nt = n // _TM
    return pl.pallas_call(
        _dist_argmin_body,
        grid=(nt,),
        in_specs=[
            pl.BlockSpec((_TM, _D), lambda t: (t, 0)),
            pl.BlockSpec((_K, _D), lambda t: (0, 0)),
            pl.BlockSpec((1, _K // 2), lambda t: (0, 0)),
            pl.BlockSpec((1, _K), lambda t: (0, 0)),
        ],
        out_specs=[
            pl.BlockSpec((1, 1, _TM), lambda t: (t, 0, 0)),
            pl.BlockSpec((1, 1), lambda t: (0, 0)),
        ],
        out_shape=[
            jax.ShapeDtypeStruct((nt, 1, _TM), jnp.int32),
            jax.ShapeDtypeStruct((1, 1), jnp.float32),
        ],
    )(z, w, jnp.arange(_K // 2, dtype=jnp.float32).reshape(1, -1), _w2_row(w))


@functools.lru_cache(maxsize=None)
def _make_sc_gather(b, d):
    info = plsc.get_sparse_core_info()
    nw = info.num_cores * info.num_subcores
    b_per_w = b // nw
    mesh = plsc.VectorSubcoreMesh(core_axis_name="c", subcore_axis_name="s")

    @functools.partial(
        pl.kernel, mesh=mesh,
        out_type=jax.ShapeDtypeStruct((b, d), jnp.float32),
        compiler_params=pltpu.CompilerParams(use_tc_tiling_on_sc=False),
        scratch_types=[
            pltpu.VMEM((b_per_w,), jnp.int32),
            pltpu.VMEM((b_per_w, d), jnp.float32),
            pltpu.SemaphoreType.DMA,
        ],
    )
    def gather_k(table_hbm, idx_hbm, out_hbm, idx_v, rows_v, sem):
        wid = lax.axis_index("s") * info.num_cores + lax.axis_index("c")
        base = wid * b_per_w
        pltpu.sync_copy(idx_hbm.at[pl.ds(base, b_per_w)], idx_v)
        pltpu.async_copy(table_hbm.at[idx_v], rows_v, sem).wait()
        pltpu.sync_copy(rows_v, out_hbm.at[pl.ds(base, b_per_w)])

    return gather_k


def kernel(x, W):
    b, c, h, w = x.shape
    xp = jnp.transpose(x, (0, 2, 3, 1))
    z = xp.reshape(-1, c)
    idx3, loss11 = _dist_argmin(z, W)
    idx = idx3.reshape(-1)
    embed = _make_sc_gather(z.shape[0], c)(W, idx)
    embed_out = jnp.transpose(embed.reshape(b, h, w, c), (0, 3, 1, 2))
    return embed_out, loss11[0, 0], idx
